# baseline (speedup 1.0000x reference)
import functools

import jax
import jax.numpy as jnp
from jax import lax
from jax.experimental import pallas as pl
from jax.experimental.pallas import tpu as pltpu
from jax.experimental.pallas import tpu_sc as plsc

H = W = 2048
NCIN = 10
NCOUT = 8

_info = plsc.get_sparse_core_info()
NC, NS, L = _info.num_cores, _info.num_subcores, _info.num_lanes
NW = NC * NS
ROWS_PW = H // NW
GROUPS = W // 16


def _mag(dx, dy):
    ss = dx * dx + dy * dy
    i = lax.bitcast_convert_type(ss, jnp.int32)
    r = lax.bitcast_convert_type(jnp.int32(0x5F3759DF) - (i >> 1), jnp.float32)
    hs = 0.5 * ss
    for _ in range(2):
        r = r * (1.5 - hs * (r * r))
    return ss * r


def _compute_chunk(in_v, mag_v, out_v):
    def mag_body(g, carry):
        s = pl.ds(g * 16, 16)
        mag_v[s] = _mag(in_v[8, s], in_v[9, s])
        return carry

    lax.fori_loop(0, GROUPS, mag_body, 0, unroll=8)

    def sel_body(g, carry):
        s = pl.ds(g * 16, 16)
        c = [in_v[j, s] for j in range(NCOUT)]
        m01, m23 = jnp.maximum(c[0], c[1]), jnp.maximum(c[2], c[3])
        m45, m67 = jnp.maximum(c[4], c[5]), jnp.maximum(c[6], c[7])
        best = jnp.maximum(jnp.maximum(m01, m23), jnp.maximum(m45, m67))
        mag = mag_v[s]
        for b in range(NCOUT):
            out_v[b, s] = jnp.where(c[b] == best, mag, 0.0)
        return carry

    lax.fori_loop(0, GROUPS, sel_body, 0, unroll=8)


@functools.partial(
    pl.kernel,
    out_type=jax.ShapeDtypeStruct((1, NCOUT, H, W), jnp.float32),
    mesh=plsc.VectorSubcoreMesh(core_axis_name="c", subcore_axis_name="s"),
    scratch_types=[
        pltpu.VMEM((NCIN, W), jnp.float32),
        pltpu.VMEM((NCIN, W), jnp.float32),
        pltpu.VMEM((W,), jnp.float32),
        pltpu.VMEM((NCOUT, W), jnp.float32),
        pltpu.VMEM((NCOUT, W), jnp.float32),
        pltpu.SemaphoreType.DMA,
        pltpu.SemaphoreType.DMA,
        pltpu.SemaphoreType.DMA,
        pltpu.SemaphoreType.DMA,
    ],
)
def _hist_sc(x_hbm, out_hbm, in_v0, in_v1, mag_v, out_v0, out_v1,
             isem0, isem1, osem0, osem1):
    wid = lax.axis_index("s") * NC + lax.axis_index("c")
    row0 = wid * ROWS_PW

    in_bufs = (in_v0, in_v1)
    out_bufs = (out_v0, out_v1)
    isems = (isem0, isem1)
    osems = (osem0, osem1)

    def start_in(ci, b):
        pltpu.async_copy(x_hbm.at[0, :, row0 + ci, :], in_bufs[b], isems[b])

    def wait_in(b):
        pltpu.make_async_copy(x_hbm.at[0, :, row0, :], in_bufs[b], isems[b]).wait()

    def start_out(ci, b):
        pltpu.async_copy(out_bufs[b], out_hbm.at[0, :, row0 + ci, :], osems[b])

    def wait_out(b):
        pltpu.make_async_copy(out_bufs[b], out_hbm.at[0, :, row0, :], osems[b]).wait()

    start_in(0, 0)
    start_in(1, 1)
    for b in range(2):
        wait_in(b)
        _compute_chunk(in_bufs[b], mag_v, out_bufs[b])
        start_out(b, b)
        start_in(b + 2, b)

    def pair(p, carry):
        for b in range(2):
            ci = 2 * p + b
            wait_in(b)
            wait_out(b)
            _compute_chunk(in_bufs[b], mag_v, out_bufs[b])
            start_out(ci, b)
            start_in(ci + 2, b)
        return carry

    lax.fori_loop(1, ROWS_PW // 2 - 1, pair, 0)

    for b in range(2):
        ci = ROWS_PW - 2 + b
        wait_in(b)
        wait_out(b)
        _compute_chunk(in_bufs[b], mag_v, out_bufs[b])
        start_out(ci, b)
    for b in range(2):
        wait_out(b)


def kernel(x):
    return _hist_sc(x)


# speedup vs baseline: 2.0704x; 2.0704x over previous
import functools

import jax
import jax.numpy as jnp
from jax import lax
from jax.experimental import pallas as pl
from jax.experimental.pallas import tpu as pltpu
from jax.experimental.pallas import tpu_sc as plsc

H = W = 2048
NCIN = 10
NCOUT = 8

_info = plsc.get_sparse_core_info()
NC, NS, L = _info.num_cores, _info.num_subcores, _info.num_lanes
NW = NC * NS
ROWS_PW = H // NW
GROUPS = W // 16


def _mag(dx, dy):
    ss = dx * dx + dy * dy
    i = lax.bitcast_convert_type(ss, jnp.int32)
    r = lax.bitcast_convert_type(jnp.int32(0x5F3759DF) - (i >> 1), jnp.float32)
    hs = 0.5 * ss
    for _ in range(2):
        r = r * (1.5 - hs * (r * r))
    return ss * r


def _compute_chunk(in_v, out_v):
    def body(g, carry):
        s = pl.ds(g * 16, 16)
        c = [in_v[j, s] for j in range(NCOUT)]
        m01, m23 = jnp.maximum(c[0], c[1]), jnp.maximum(c[2], c[3])
        m45, m67 = jnp.maximum(c[4], c[5]), jnp.maximum(c[6], c[7])
        best = jnp.maximum(jnp.maximum(m01, m23), jnp.maximum(m45, m67))
        mag = _mag(in_v[8, s], in_v[9, s])
        for b in range(NCOUT):
            out_v[b, s] = jnp.where(c[b] == best, mag, 0.0)
        return carry

    lax.fori_loop(0, GROUPS, body, 0, unroll=8)


@functools.partial(
    pl.kernel,
    out_type=jax.ShapeDtypeStruct((1, NCOUT, H, W), jnp.float32),
    mesh=plsc.VectorSubcoreMesh(core_axis_name="c", subcore_axis_name="s"),
    scratch_types=[
        pltpu.VMEM((NCIN, W), jnp.float32),
        pltpu.VMEM((NCIN, W), jnp.float32),
        pltpu.VMEM((NCOUT, W), jnp.float32),
        pltpu.VMEM((NCOUT, W), jnp.float32),
        pltpu.SemaphoreType.DMA,
        pltpu.SemaphoreType.DMA,
        pltpu.SemaphoreType.DMA,
        pltpu.SemaphoreType.DMA,
    ],
)
def _hist_sc(x_hbm, out_hbm, in_v0, in_v1, out_v0, out_v1,
             isem0, isem1, osem0, osem1):
    wid = lax.axis_index("s") * NC + lax.axis_index("c")
    row0 = wid * ROWS_PW

    in_bufs = (in_v0, in_v1)
    out_bufs = (out_v0, out_v1)
    isems = (isem0, isem1)
    osems = (osem0, osem1)

    def start_in(ci, b):
        pltpu.async_copy(x_hbm.at[0, :, row0 + ci, :], in_bufs[b], isems[b])

    def wait_in(b):
        pltpu.make_async_copy(x_hbm.at[0, :, row0, :], in_bufs[b], isems[b]).wait()

    def start_out(ci, b):
        pltpu.async_copy(out_bufs[b], out_hbm.at[0, :, row0 + ci, :], osems[b])

    def wait_out(b):
        pltpu.make_async_copy(out_bufs[b], out_hbm.at[0, :, row0, :], osems[b]).wait()

    start_in(0, 0)
    start_in(1, 1)
    for b in range(2):
        wait_in(b)
        _compute_chunk(in_bufs[b], out_bufs[b])
        start_out(b, b)
        start_in(b + 2, b)

    def pair(p, carry):
        for b in range(2):
            ci = 2 * p + b
            wait_in(b)
            wait_out(b)
            _compute_chunk(in_bufs[b], out_bufs[b])
            start_out(ci, b)
            start_in(ci + 2, b)
        return carry

    lax.fori_loop(1, ROWS_PW // 2 - 1, pair, 0)

    for b in range(2):
        ci = ROWS_PW - 2 + b
        wait_in(b)
        wait_out(b)
        _compute_chunk(in_bufs[b], out_bufs[b])
        start_out(ci, b)
    for b in range(2):
        wait_out(b)


def kernel(x):
    return _hist_sc(x)
